# Initial kernel scaffold; baseline (speedup 1.0000x reference)
import jax
import jax.numpy as jnp
from jax.experimental import pallas as pl


def kernel(x, edge_index, edge_attr, batch, We0, be0, Wx0, W20, b20, We1, be1, Wx1, W21, b21, We2, be2, Wx2, W22, b22, lin_W, lin_b):
    raise NotImplementedError("write your pallas kernel here")


# speedup vs baseline: 2.0057x; 2.0057x over previous
import functools

import jax
import jax.numpy as jnp
from jax import lax
from jax.experimental import pallas as pl
from jax.experimental.pallas import tpu as pltpu
from jax.experimental.pallas import tpu_sc as plsc

N = 10000
E = 320000
H = 128
NCORES = 2
NSUB = 16
NW = NCORES * NSUB
EPW = E // NW
KCH = 40
NCHUNK = EPW // KCH
RPT = 624
TAIL0 = RPT * NSUB
TAILN = N - TAIL0
EB = 4000


def _proj_body(x_ref, wsrc_ref, wdst_ref, gsrc_ref, gdst_ref):
    x = x_ref[...]
    gsrc_ref[...] = jnp.dot(x, wsrc_ref[...], preferred_element_type=jnp.float32)
    gdst_ref[...] = jnp.dot(x, wdst_ref[...], preferred_element_type=jnp.float32)


def _proj(x, wsrc, wdst):
    return pl.pallas_call(
        _proj_body,
        out_shape=[jax.ShapeDtypeStruct((N, 2 * H), jnp.float32),
                   jax.ShapeDtypeStruct((N, H), jnp.float32)],
    )(x, wsrc, wdst)


def _edgemm_body(e_ref, w_ref, b_ref, out_ref):
    out_ref[...] = jnp.dot(e_ref[...], w_ref[...],
                           preferred_element_type=jnp.float32) + b_ref[...]


def _edgemm(ea, wc, be):
    kd = ea.shape[1]
    return pl.pallas_call(
        _edgemm_body,
        grid=(E // EB,),
        in_specs=[pl.BlockSpec((EB, kd), lambda i: (i, 0)),
                  pl.BlockSpec((kd, H), lambda i: (0, 0)),
                  pl.BlockSpec((1, H), lambda i: (0, 0))],
        out_specs=pl.BlockSpec((EB, H), lambda i: (i, 0)),
        out_shape=jax.ShapeDtypeStruct((E, H), jnp.float32),
    )(ea, wc, be)


def _post_body(xp_ref, agg_ref, w2_ref, b2_ref, xnew_ref, mean_ref):
    agg = agg_ref[:N] + agg_ref[N:]
    xn = jnp.maximum(
        jnp.dot(xp_ref[...] + agg, w2_ref[...],
                preferred_element_type=jnp.float32) + b2_ref[...], 0.0)
    xnew_ref[...] = xn
    mean_ref[...] = jnp.sum(xn, axis=0, keepdims=True) * (1.0 / N)


def _post(gsrc, agg, w2, b2):
    return pl.pallas_call(
        _post_body,
        grid=(1,),
        in_specs=[pl.BlockSpec((N, H), lambda i: (0, 1)),
                  pl.BlockSpec((2 * N, H), lambda i: (0, 0)),
                  pl.BlockSpec((H, H), lambda i: (0, 0)),
                  pl.BlockSpec((1, H), lambda i: (0, 0))],
        out_specs=[pl.BlockSpec((N, H), lambda i: (0, 0)),
                   pl.BlockSpec((1, H), lambda i: (0, 0))],
        out_shape=[jax.ShapeDtypeStruct((N, H), jnp.float32),
                   jax.ShapeDtypeStruct((1, H), jnp.float32)],
    )(gsrc, agg, w2, b2)


def _final_body(xp_ref, agg_ref, w2_ref, b2_ref, m0_ref, m1_ref, lw_ref,
                lb_ref, out_ref):
    agg = agg_ref[:N] + agg_ref[N:]
    xn = jnp.maximum(
        jnp.dot(xp_ref[...] + agg, w2_ref[...],
                preferred_element_type=jnp.float32) + b2_ref[...], 0.0)
    m2 = jnp.sum(xn, axis=0, keepdims=True) * (1.0 / N)
    acc = (jnp.sum(m0_ref[...] * lw_ref[0, :]) +
           jnp.sum(m1_ref[...] * lw_ref[1, :]) +
           jnp.sum(m2 * lw_ref[2, :]))
    out_ref[...] = acc + lb_ref[...]


def _final(gsrc, agg, w2, b2, m0, m1, lw, lb):
    return pl.pallas_call(
        _final_body,
        grid=(1,),
        in_specs=[pl.BlockSpec((N, H), lambda i: (0, 1)),
                  pl.BlockSpec((2 * N, H), lambda i: (0, 0)),
                  pl.BlockSpec((H, H), lambda i: (0, 0)),
                  pl.BlockSpec((1, H), lambda i: (0, 0)),
                  pl.BlockSpec((1, H), lambda i: (0, 0)),
                  pl.BlockSpec((1, H), lambda i: (0, 0)),
                  pl.BlockSpec((3, H), lambda i: (0, 0)),
                  pl.BlockSpec((1, 1), lambda i: (0, 0))],
        out_specs=pl.BlockSpec((1, 1), lambda i: (0, 0)),
        out_shape=jax.ShapeDtypeStruct((1, 1), jnp.float32),
    )(gsrc, agg, w2, b2, m0, m1, lw, lb)


def _make_sc_edge(write_e: bool):
    mesh = plsc.VectorSubcoreMesh(core_axis_name="c", subcore_axis_name="s")
    out_type = []
    if write_e:
        out_type.append(jax.ShapeDtypeStruct((E, H), jnp.float32))
    out_type.append(jax.ShapeDtypeStruct((NCORES * N, H), jnp.float32))
    scratch = [
        pltpu.VMEM((KCH,), jnp.int32),
        pltpu.VMEM((KCH,), jnp.int32),
        pltpu.VMEM((KCH, 2 * H), jnp.float32),
        pltpu.VMEM((KCH, H), jnp.float32),
        pltpu.VMEM((KCH, H), jnp.float32),
        pltpu.VMEM((KCH, H), jnp.float32),
        pltpu.VMEM((KCH, H), jnp.float32),
        pltpu.VMEM_SHARED((N, H), jnp.float32),
        pltpu.SemaphoreType.DMA,
        pltpu.SemaphoreType.DMA,
    ]

    def body(gsrc, gdst, cterm, srci, dsti, zeros, *rest):
        if write_e:
            e_hbm, agg_hbm = rest[0], rest[1]
            rest = rest[2:]
        else:
            agg_hbm = rest[0]
            rest = rest[1:]
        si, di, g, d, c, ev, mv, aggsh, sem1, sem2 = rest
        cid = lax.axis_index("c")
        sid = lax.axis_index("s")
        wid = sid * NCORES + cid
        r0 = sid * RPT

        pltpu.sync_copy(zeros.at[pl.ds(r0, RPT)], aggsh.at[pl.ds(r0, RPT)])

        @pl.when(sid == NSUB - 1)
        def _():
            pltpu.sync_copy(zeros.at[pl.ds(TAIL0, TAILN)],
                            aggsh.at[pl.ds(TAIL0, TAILN)])

        plsc.subcore_barrier()

        base0 = wid * EPW

        def chunk(i, carry):
            base = base0 + i * KCH
            pltpu.sync_copy(srci.at[pl.ds(base, KCH)], si)
            pltpu.sync_copy(dsti.at[pl.ds(base, KCH)], di)
            cp_g = pltpu.async_copy(gsrc.at[si], g, sem1)
            cp_d = pltpu.async_copy(gdst.at[di], d, sem2)
            pltpu.sync_copy(cterm.at[pl.ds(base, KCH)], c)
            cp_g.wait()
            cp_d.wait()

            def row(j, carry2):
                for q in range(8):
                    sl = pl.ds(q * 16, 16)
                    e = jnp.maximum(g[j, sl] + d[j, sl] + c[j, sl], 0.0)
                    m = jnp.maximum(g[j, pl.ds(H + q * 16, 16)] + e, 0.0)
                    ev[j, sl] = e
                    mv[j, sl] = m
                return carry2

            lax.fori_loop(0, KCH, row, 0)
            if write_e:
                pltpu.sync_copy(ev, e_hbm.at[pl.ds(base, KCH)])
            pltpu.sync_copy(mv, aggsh.at[di], add=True)
            return carry

        lax.fori_loop(0, NCHUNK, chunk, 0)
        plsc.subcore_barrier()

        o0 = cid * N
        pltpu.sync_copy(aggsh.at[pl.ds(r0, RPT)], agg_hbm.at[pl.ds(o0 + r0, RPT)])

        @pl.when(sid == NSUB - 1)
        def _():
            pltpu.sync_copy(aggsh.at[pl.ds(TAIL0, TAILN)],
                            agg_hbm.at[pl.ds(o0 + TAIL0, TAILN)])

    return pl.kernel(body, out_type=out_type, mesh=mesh, scratch_types=scratch)


_sc_edge_we = _make_sc_edge(True)
_sc_edge_noe = _make_sc_edge(False)


def kernel(x, edge_index, edge_attr, batch, We0, be0, Wx0, W20, b20,
           We1, be1, Wx1, W21, b21, We2, be2, Wx2, W22, b22, lin_W, lin_b):
    src = edge_index[0]
    dst = edge_index[1]
    zeros = jnp.zeros((N, H), jnp.float32)
    params = [(We0, be0, Wx0, W20, b20), (We1, be1, Wx1, W21, b21),
              (We2, be2, Wx2, W22, b22)]

    ea = edge_attr
    means = []
    out = None
    for b, (We, be, Wx, W2, b2) in enumerate(params):
        in_x = H
        Wa = We[:in_x]
        Wb = We[in_x:2 * in_x]
        Wc = We[2 * in_x:]
        wsrc = jnp.concatenate([Wa, Wx], axis=1)
        gsrc, gdst = _proj(x, wsrc, Wb)
        C = _edgemm(ea, Wc, be.reshape(1, H))
        if b < 2:
            e_new, agg = _sc_edge_we(gsrc, gdst, C, src, dst, zeros)
            xnew, mean = _post(gsrc, agg, W2, b2.reshape(1, H))
            x = xnew
            ea = e_new
            means.append(mean)
        else:
            (agg,) = _sc_edge_noe(gsrc, gdst, C, src, dst, zeros)
            lw = lin_W.reshape(3, H)
            out = _final(gsrc, agg, W22, b22.reshape(1, H),
                         means[0], means[1], lw, lin_b.reshape(1, 1))
    return out


# speedup vs baseline: 2.0607x; 1.0274x over previous
import functools

import jax
import jax.numpy as jnp
from jax import lax
from jax.experimental import pallas as pl
from jax.experimental.pallas import tpu as pltpu
from jax.experimental.pallas import tpu_sc as plsc

N = 10000
E = 320000
H = 128
NCORES = 2
NSUB = 16
NW = NCORES * NSUB
EPW = E // NW
KCH = 40
NCHUNK = EPW // KCH
RPT = 624
TAIL0 = RPT * NSUB
TAILN = N - TAIL0
EB = 4000


def _proj_body(x_ref, wsrc_ref, wdst_ref, gsrc_ref, gdst_ref):
    x = x_ref[...]
    gsrc_ref[...] = jnp.dot(x, wsrc_ref[...], preferred_element_type=jnp.float32)
    gdst_ref[...] = jnp.dot(x, wdst_ref[...], preferred_element_type=jnp.float32)


def _proj(x, wsrc, wdst):
    return pl.pallas_call(
        _proj_body,
        out_shape=[jax.ShapeDtypeStruct((N, 2 * H), jnp.float32),
                   jax.ShapeDtypeStruct((N, H), jnp.float32)],
    )(x, wsrc, wdst)


def _edgemm_body(e_ref, w_ref, b_ref, out_ref):
    out_ref[...] = jnp.dot(e_ref[...], w_ref[...],
                           preferred_element_type=jnp.float32) + b_ref[...]


def _edgemm(ea, wc, be):
    kd = ea.shape[1]
    return pl.pallas_call(
        _edgemm_body,
        grid=(E // EB,),
        in_specs=[pl.BlockSpec((EB, kd), lambda i: (i, 0)),
                  pl.BlockSpec((kd, H), lambda i: (0, 0)),
                  pl.BlockSpec((1, H), lambda i: (0, 0))],
        out_specs=pl.BlockSpec((EB, H), lambda i: (i, 0)),
        out_shape=jax.ShapeDtypeStruct((E, H), jnp.float32),
    )(ea, wc, be)


def _post_body(xp_ref, agg_ref, w2_ref, b2_ref, xnew_ref, mean_ref):
    agg = agg_ref[:N] + agg_ref[N:]
    xn = jnp.maximum(
        jnp.dot(xp_ref[...] + agg, w2_ref[...],
                preferred_element_type=jnp.float32) + b2_ref[...], 0.0)
    xnew_ref[...] = xn
    mean_ref[...] = jnp.sum(xn, axis=0, keepdims=True) * (1.0 / N)


def _post(gsrc, agg, w2, b2):
    return pl.pallas_call(
        _post_body,
        grid=(1,),
        in_specs=[pl.BlockSpec((N, H), lambda i: (0, 1)),
                  pl.BlockSpec((2 * N, H), lambda i: (0, 0)),
                  pl.BlockSpec((H, H), lambda i: (0, 0)),
                  pl.BlockSpec((1, H), lambda i: (0, 0))],
        out_specs=[pl.BlockSpec((N, H), lambda i: (0, 0)),
                   pl.BlockSpec((1, H), lambda i: (0, 0))],
        out_shape=[jax.ShapeDtypeStruct((N, H), jnp.float32),
                   jax.ShapeDtypeStruct((1, H), jnp.float32)],
    )(gsrc, agg, w2, b2)


def _final_body(xp_ref, agg_ref, w2_ref, b2_ref, m0_ref, m1_ref, lw_ref,
                lb_ref, out_ref):
    agg = agg_ref[:N] + agg_ref[N:]
    xn = jnp.maximum(
        jnp.dot(xp_ref[...] + agg, w2_ref[...],
                preferred_element_type=jnp.float32) + b2_ref[...], 0.0)
    m2 = jnp.sum(xn, axis=0, keepdims=True) * (1.0 / N)
    acc = (jnp.sum(m0_ref[...] * lw_ref[0, :]) +
           jnp.sum(m1_ref[...] * lw_ref[1, :]) +
           jnp.sum(m2 * lw_ref[2, :]))
    out_ref[...] = acc + lb_ref[...]


def _final(gsrc, agg, w2, b2, m0, m1, lw, lb):
    return pl.pallas_call(
        _final_body,
        grid=(1,),
        in_specs=[pl.BlockSpec((N, H), lambda i: (0, 1)),
                  pl.BlockSpec((2 * N, H), lambda i: (0, 0)),
                  pl.BlockSpec((H, H), lambda i: (0, 0)),
                  pl.BlockSpec((1, H), lambda i: (0, 0)),
                  pl.BlockSpec((1, H), lambda i: (0, 0)),
                  pl.BlockSpec((1, H), lambda i: (0, 0)),
                  pl.BlockSpec((3, H), lambda i: (0, 0)),
                  pl.BlockSpec((1, 1), lambda i: (0, 0))],
        out_specs=pl.BlockSpec((1, 1), lambda i: (0, 0)),
        out_shape=jax.ShapeDtypeStruct((1, 1), jnp.float32),
    )(gsrc, agg, w2, b2, m0, m1, lw, lb)


def _make_sc_edge(write_e: bool):
    mesh = plsc.VectorSubcoreMesh(core_axis_name="c", subcore_axis_name="s")
    out_type = []
    if write_e:
        out_type.append(jax.ShapeDtypeStruct((E, H), jnp.float32))
    out_type.append(jax.ShapeDtypeStruct((NCORES * N, H), jnp.float32))
    scratch = [
        pltpu.VMEM((KCH,), jnp.int32),
        pltpu.VMEM((KCH,), jnp.int32),
        pltpu.VMEM((KCH,), jnp.int32),
        pltpu.VMEM((KCH,), jnp.int32),
        pltpu.VMEM((KCH, 2 * H), jnp.float32),
        pltpu.VMEM((KCH, 2 * H), jnp.float32),
        pltpu.VMEM((KCH, H), jnp.float32),
        pltpu.VMEM((KCH, H), jnp.float32),
        pltpu.VMEM((KCH, H), jnp.float32),
        pltpu.VMEM((KCH, H), jnp.float32),
        pltpu.VMEM((KCH, H), jnp.float32),
        pltpu.VMEM_SHARED((N, H), jnp.float32),
        pltpu.SemaphoreType.DMA,
        pltpu.SemaphoreType.DMA,
        pltpu.SemaphoreType.DMA,
        pltpu.SemaphoreType.DMA,
    ]

    def body(gsrc, gdst, cterm, srci, dsti, zeros, *rest):
        if write_e:
            e_hbm, agg_hbm = rest[0], rest[1]
            rest = rest[2:]
        else:
            agg_hbm = rest[0]
            rest = rest[1:]
        (si0, di0, si1, di1, g0, g1, d0, d1, c, ev, mv, aggsh,
         sg0, sg1, sd0, sd1) = rest
        cid = lax.axis_index("c")
        sid = lax.axis_index("s")
        wid = sid * NCORES + cid
        r0 = sid * RPT

        pltpu.sync_copy(zeros.at[pl.ds(r0, RPT)], aggsh.at[pl.ds(r0, RPT)])

        @pl.when(sid == NSUB - 1)
        def _():
            pltpu.sync_copy(zeros.at[pl.ds(TAIL0, TAILN)],
                            aggsh.at[pl.ds(TAIL0, TAILN)])

        plsc.subcore_barrier()

        base0 = wid * EPW
        slots = ((si0, di0, g0, d0, sg0, sd0), (si1, di1, g1, d1, sg1, sd1))

        def issue(slot, base):
            si_, di_, g_, d_, sg_, sd_ = slot
            pltpu.sync_copy(srci.at[pl.ds(base, KCH)], si_)
            pltpu.sync_copy(dsti.at[pl.ds(base, KCH)], di_)
            pltpu.async_copy(gsrc.at[si_], g_, sg_)
            pltpu.async_copy(gdst.at[di_], d_, sd_)

        def process(slot, base):
            si_, di_, g_, d_, sg_, sd_ = slot
            pltpu.sync_copy(cterm.at[pl.ds(base, KCH)], c)
            pltpu.make_async_copy(gsrc.at[si_], g_, sg_).wait()
            pltpu.make_async_copy(gdst.at[di_], d_, sd_).wait()

            def row(j, carry2):
                for q in range(8):
                    sl = pl.ds(q * 16, 16)
                    e = jnp.maximum(g_[j, sl] + d_[j, sl] + c[j, sl], 0.0)
                    m = jnp.maximum(g_[j, pl.ds(H + q * 16, 16)] + e, 0.0)
                    ev[j, sl] = e
                    mv[j, sl] = m
                return carry2

            lax.fori_loop(0, KCH, row, 0)
            if write_e:
                pltpu.sync_copy(ev, e_hbm.at[pl.ds(base, KCH)])
            pltpu.sync_copy(mv, aggsh.at[di_], add=True)

        issue(slots[0], base0)

        def dbl(it, carry):
            i0 = it * 2
            issue(slots[1], base0 + (i0 + 1) * KCH)
            process(slots[0], base0 + i0 * KCH)

            @pl.when(i0 + 2 < NCHUNK)
            def _():
                issue(slots[0], base0 + (i0 + 2) * KCH)

            process(slots[1], base0 + (i0 + 1) * KCH)
            return carry

        lax.fori_loop(0, NCHUNK // 2, dbl, 0)
        plsc.subcore_barrier()

        o0 = cid * N
        pltpu.sync_copy(aggsh.at[pl.ds(r0, RPT)], agg_hbm.at[pl.ds(o0 + r0, RPT)])

        @pl.when(sid == NSUB - 1)
        def _():
            pltpu.sync_copy(aggsh.at[pl.ds(TAIL0, TAILN)],
                            agg_hbm.at[pl.ds(o0 + TAIL0, TAILN)])

    return pl.kernel(body, out_type=out_type, mesh=mesh, scratch_types=scratch)


_sc_edge_we = _make_sc_edge(True)
_sc_edge_noe = _make_sc_edge(False)


def kernel(x, edge_index, edge_attr, batch, We0, be0, Wx0, W20, b20,
           We1, be1, Wx1, W21, b21, We2, be2, Wx2, W22, b22, lin_W, lin_b):
    src = edge_index[0]
    dst = edge_index[1]
    zeros = jnp.zeros((N, H), jnp.float32)
    params = [(We0, be0, Wx0, W20, b20), (We1, be1, Wx1, W21, b21),
              (We2, be2, Wx2, W22, b22)]

    ea = edge_attr
    means = []
    out = None
    for b, (We, be, Wx, W2, b2) in enumerate(params):
        in_x = H
        Wa = We[:in_x]
        Wb = We[in_x:2 * in_x]
        Wc = We[2 * in_x:]
        wsrc = jnp.concatenate([Wa, Wx], axis=1)
        gsrc, gdst = _proj(x, wsrc, Wb)
        C = _edgemm(ea, Wc, be.reshape(1, H))
        if b < 2:
            e_new, agg = _sc_edge_we(gsrc, gdst, C, src, dst, zeros)
            xnew, mean = _post(gsrc, agg, W2, b2.reshape(1, H))
            x = xnew
            ea = e_new
            means.append(mean)
        else:
            (agg,) = _sc_edge_noe(gsrc, gdst, C, src, dst, zeros)
            lw = lin_W.reshape(3, H)
            out = _final(gsrc, agg, W22, b22.reshape(1, H),
                         means[0], means[1], lw, lin_b.reshape(1, 1))
    return out


# speedup vs baseline: 3.1332x; 1.5205x over previous
import functools

import jax
import jax.numpy as jnp
from jax import lax
from jax.experimental import pallas as pl
from jax.experimental.pallas import tpu as pltpu
from jax.experimental.pallas import tpu_sc as plsc

N = 10000
E = 320000
H = 128
NCORES = 2
NSUB = 16
NW = NCORES * NSUB
EPW = E // NW
KCH = 40
NCHUNK = EPW // KCH
RPT = 624
TAIL0 = RPT * NSUB
TAILN = N - TAIL0
EB = 4000


def _proj_body(x_ref, wsrc_ref, wdst_ref, gsrc_ref, gdst_ref):
    x = x_ref[...]
    gsrc_ref[...] = jnp.dot(x, wsrc_ref[...], preferred_element_type=jnp.float32)
    gdst_ref[...] = jnp.dot(x, wdst_ref[...], preferred_element_type=jnp.float32)


def _proj(x, wsrc, wdst):
    return pl.pallas_call(
        _proj_body,
        out_shape=[jax.ShapeDtypeStruct((N, 2 * H), jnp.float32),
                   jax.ShapeDtypeStruct((N, H), jnp.float32)],
    )(x, wsrc, wdst)


def _edgemm_body(e_ref, w_ref, b_ref, out_ref):
    out_ref[...] = jnp.dot(e_ref[...], w_ref[...],
                           preferred_element_type=jnp.float32) + b_ref[...]


def _edgemm(ea, wc, be):
    kd = ea.shape[1]
    return pl.pallas_call(
        _edgemm_body,
        grid=(E // EB,),
        in_specs=[pl.BlockSpec((EB, kd), lambda i: (i, 0)),
                  pl.BlockSpec((kd, H), lambda i: (0, 0)),
                  pl.BlockSpec((1, H), lambda i: (0, 0))],
        out_specs=pl.BlockSpec((EB, H), lambda i: (i, 0)),
        out_shape=jax.ShapeDtypeStruct((E, H), jnp.float32),
    )(ea, wc, be)


def _post_body(xp_ref, agg_ref, w2_ref, b2_ref, xnew_ref, mean_ref):
    agg = agg_ref[:N] + agg_ref[N:]
    xn = jnp.maximum(
        jnp.dot(xp_ref[...] + agg, w2_ref[...],
                preferred_element_type=jnp.float32) + b2_ref[...], 0.0)
    xnew_ref[...] = xn
    mean_ref[...] = jnp.sum(xn, axis=0, keepdims=True) * (1.0 / N)


def _post(gsrc, agg, w2, b2):
    return pl.pallas_call(
        _post_body,
        grid=(1,),
        in_specs=[pl.BlockSpec((N, H), lambda i: (0, 1)),
                  pl.BlockSpec((2 * N, H), lambda i: (0, 0)),
                  pl.BlockSpec((H, H), lambda i: (0, 0)),
                  pl.BlockSpec((1, H), lambda i: (0, 0))],
        out_specs=[pl.BlockSpec((N, H), lambda i: (0, 0)),
                   pl.BlockSpec((1, H), lambda i: (0, 0))],
        out_shape=[jax.ShapeDtypeStruct((N, H), jnp.float32),
                   jax.ShapeDtypeStruct((1, H), jnp.float32)],
    )(gsrc, agg, w2, b2)


def _final_body(xp_ref, agg_ref, w2_ref, b2_ref, m0_ref, m1_ref, lw_ref,
                lb_ref, out_ref):
    agg = agg_ref[:N] + agg_ref[N:]
    xn = jnp.maximum(
        jnp.dot(xp_ref[...] + agg, w2_ref[...],
                preferred_element_type=jnp.float32) + b2_ref[...], 0.0)
    m2 = jnp.sum(xn, axis=0, keepdims=True) * (1.0 / N)
    acc = (jnp.sum(m0_ref[...] * lw_ref[0, :]) +
           jnp.sum(m1_ref[...] * lw_ref[1, :]) +
           jnp.sum(m2 * lw_ref[2, :]))
    out_ref[...] = acc + lb_ref[...]


def _final(gsrc, agg, w2, b2, m0, m1, lw, lb):
    return pl.pallas_call(
        _final_body,
        grid=(1,),
        in_specs=[pl.BlockSpec((N, H), lambda i: (0, 1)),
                  pl.BlockSpec((2 * N, H), lambda i: (0, 0)),
                  pl.BlockSpec((H, H), lambda i: (0, 0)),
                  pl.BlockSpec((1, H), lambda i: (0, 0)),
                  pl.BlockSpec((1, H), lambda i: (0, 0)),
                  pl.BlockSpec((1, H), lambda i: (0, 0)),
                  pl.BlockSpec((3, H), lambda i: (0, 0)),
                  pl.BlockSpec((1, 1), lambda i: (0, 0))],
        out_specs=pl.BlockSpec((1, 1), lambda i: (0, 0)),
        out_shape=jax.ShapeDtypeStruct((1, 1), jnp.float32),
    )(gsrc, agg, w2, b2, m0, m1, lw, lb)


def _make_sc_edge(write_e: bool):
    mesh = plsc.VectorSubcoreMesh(core_axis_name="c", subcore_axis_name="s")
    out_type = []
    if write_e:
        out_type.append(jax.ShapeDtypeStruct((E, H), jnp.float32))
    out_type.append(jax.ShapeDtypeStruct((NCORES * N, H), jnp.float32))
    scratch = [
        pltpu.VMEM((KCH,), jnp.int32),
        pltpu.VMEM((KCH,), jnp.int32),
        pltpu.VMEM((KCH,), jnp.int32),
        pltpu.VMEM((KCH,), jnp.int32),
        pltpu.VMEM((KCH, 2 * H), jnp.float32),
        pltpu.VMEM((KCH, 2 * H), jnp.float32),
        pltpu.VMEM((KCH, H), jnp.float32),
        pltpu.VMEM((KCH, H), jnp.float32),
        pltpu.VMEM((KCH, H), jnp.float32),
        pltpu.VMEM((KCH, H), jnp.float32),
        pltpu.VMEM((KCH, H), jnp.float32),
        pltpu.VMEM_SHARED((N, H), jnp.float32),
        pltpu.SemaphoreType.DMA,
        pltpu.SemaphoreType.DMA,
        pltpu.SemaphoreType.DMA,
        pltpu.SemaphoreType.DMA,
        pltpu.SemaphoreType.DMA,
        pltpu.SemaphoreType.DMA,
    ]

    def body(gsrc, gdst, cterm, srci, dsti, zeros, *rest):
        if write_e:
            e_hbm, agg_hbm = rest[0], rest[1]
            rest = rest[2:]
        else:
            agg_hbm = rest[0]
            rest = rest[1:]
        (si0, di0, si1, di1, g0, g1, d0, d1, c, ev0, ev1, aggsh,
         sg0, sg1, sd0, sd1, se0, se1) = rest
        cid = lax.axis_index("c")
        sid = lax.axis_index("s")
        wid = sid * NCORES + cid
        r0 = sid * RPT

        pltpu.sync_copy(zeros.at[pl.ds(r0, RPT)], aggsh.at[pl.ds(r0, RPT)])

        @pl.when(sid == NSUB - 1)
        def _():
            pltpu.sync_copy(zeros.at[pl.ds(TAIL0, TAILN)],
                            aggsh.at[pl.ds(TAIL0, TAILN)])

        plsc.subcore_barrier()

        base0 = wid * EPW
        slots = ((si0, di0, g0, d0, ev0, sg0, sd0, se0),
                 (si1, di1, g1, d1, ev1, sg1, sd1, se1))

        def issue(slot, base):
            si_, di_, g_, d_, ev_, sg_, sd_, se_ = slot
            pltpu.sync_copy(srci.at[pl.ds(base, KCH)], si_)
            pltpu.sync_copy(dsti.at[pl.ds(base, KCH)], di_)
            pltpu.async_copy(gsrc.at[si_], g_, sg_)
            pltpu.async_copy(gdst.at[di_], d_, sd_)

        def process(slot, base, first):
            si_, di_, g_, d_, ev_, sg_, sd_, se_ = slot
            pltpu.sync_copy(cterm.at[pl.ds(base, KCH)], c)
            pltpu.make_async_copy(gsrc.at[si_], g_, sg_).wait()
            pltpu.make_async_copy(gdst.at[di_], d_, sd_).wait()
            if write_e and not first:
                pltpu.make_async_copy(ev_, e_hbm.at[pl.ds(base, KCH)], se_).wait()

            @plsc.parallel_loop(0, KCH, 1, unroll=2)
            def row(j):
                for q in range(8):
                    sl = pl.ds(q * 16, 16)
                    e = jnp.maximum(g_[j, sl] + d_[j, sl] + c[j, sl], 0.0)
                    m = jnp.maximum(g_[j, pl.ds(H + q * 16, 16)] + e, 0.0)
                    if write_e:
                        ev_[j, sl] = e
                    d_[j, sl] = m

            if write_e:
                pltpu.async_copy(ev_, e_hbm.at[pl.ds(base, KCH)], se_)
            pltpu.sync_copy(d_, aggsh.at[di_], add=True)

        issue(slots[0], base0)
        issue(slots[1], base0 + KCH)
        process(slots[0], base0, True)
        issue(slots[0], base0 + 2 * KCH)
        process(slots[1], base0 + KCH, True)

        def dbl(it, carry):
            i0 = it * 2
            issue(slots[1], base0 + (i0 + 1) * KCH)
            process(slots[0], base0 + i0 * KCH, False)

            @pl.when(i0 + 2 < NCHUNK)
            def _():
                issue(slots[0], base0 + (i0 + 2) * KCH)

            process(slots[1], base0 + (i0 + 1) * KCH, False)
            return carry

        lax.fori_loop(1, NCHUNK // 2, dbl, 0)
        if write_e:
            pltpu.make_async_copy(ev0, e_hbm.at[pl.ds(base0, KCH)], se0).wait()
            pltpu.make_async_copy(ev1, e_hbm.at[pl.ds(base0, KCH)], se1).wait()
        plsc.subcore_barrier()

        o0 = cid * N
        pltpu.sync_copy(aggsh.at[pl.ds(r0, RPT)], agg_hbm.at[pl.ds(o0 + r0, RPT)])

        @pl.when(sid == NSUB - 1)
        def _():
            pltpu.sync_copy(aggsh.at[pl.ds(TAIL0, TAILN)],
                            agg_hbm.at[pl.ds(o0 + TAIL0, TAILN)])

    return pl.kernel(body, out_type=out_type, mesh=mesh, scratch_types=scratch)


_sc_edge_we = _make_sc_edge(True)
_sc_edge_noe = _make_sc_edge(False)


def kernel(x, edge_index, edge_attr, batch, We0, be0, Wx0, W20, b20,
           We1, be1, Wx1, W21, b21, We2, be2, Wx2, W22, b22, lin_W, lin_b):
    src = edge_index[0]
    dst = edge_index[1]
    zeros = jnp.zeros((N, H), jnp.float32)
    params = [(We0, be0, Wx0, W20, b20), (We1, be1, Wx1, W21, b21),
              (We2, be2, Wx2, W22, b22)]

    ea = edge_attr
    means = []
    out = None
    for b, (We, be, Wx, W2, b2) in enumerate(params):
        in_x = H
        Wa = We[:in_x]
        Wb = We[in_x:2 * in_x]
        Wc = We[2 * in_x:]
        wsrc = jnp.concatenate([Wa, Wx], axis=1)
        gsrc, gdst = _proj(x, wsrc, Wb)
        C = _edgemm(ea, Wc, be.reshape(1, H))
        if b < 2:
            e_new, agg = _sc_edge_we(gsrc, gdst, C, src, dst, zeros)
            xnew, mean = _post(gsrc, agg, W2, b2.reshape(1, H))
            x = xnew
            ea = e_new
            means.append(mean)
        else:
            (agg,) = _sc_edge_noe(gsrc, gdst, C, src, dst, zeros)
            lw = lin_W.reshape(3, H)
            out = _final(gsrc, agg, W22, b22.reshape(1, H),
                         means[0], means[1], lw, lin_b.reshape(1, 1))
    return out


# speedup vs baseline: 5.5072x; 1.7577x over previous
import functools

import jax
import jax.numpy as jnp
from jax import lax
from jax.experimental import pallas as pl
from jax.experimental.pallas import tpu as pltpu
from jax.experimental.pallas import tpu_sc as plsc

N = 10000
E = 320000
H = 128
HW = H // 2
NCORES = 2
NSUB = 16
NW = NCORES * NSUB
EPW = E // NW
KCH = 40
NCHUNK = EPW // KCH
RPT = 624
TAIL0 = RPT * NSUB
TAILN = N - TAIL0
EB = 8000
SCH = 10


def _rne_hi(x):
    bits = lax.bitcast_convert_type(x, jnp.uint32)
    t = bits + jnp.uint32(0x7FFF) + ((bits >> 16) & jnp.uint32(1))
    return t & jnp.uint32(0xFFFF0000)


def _pack_words(lo, hi):
    return lax.bitcast_convert_type(_rne_hi(hi) | (_rne_hi(lo) >> 16),
                                    jnp.int32)


def _unpack_words(w):
    wu = lax.bitcast_convert_type(w, jnp.uint32)
    vlo = lax.bitcast_convert_type(wu << 16, jnp.float32)
    vhi = lax.bitcast_convert_type(wu & jnp.uint32(0xFFFF0000), jnp.float32)
    return vlo, vhi


def _proj_body(x_ref, wsrc_ref, wdst_ref, gsrc_ref, gdst_ref):
    x = x_ref[...]
    ya = jnp.dot(x, wsrc_ref[...], preferred_element_type=jnp.float32)
    gsrc_ref[...] = jnp.concatenate(
        [_pack_words(ya[:, :HW], ya[:, HW:H]),
         _pack_words(ya[:, H:H + HW], ya[:, H + HW:])], axis=1)
    gdst_ref[...] = jnp.dot(x, wdst_ref[...],
                            preferred_element_type=jnp.float32)


def _proj(x, wsrc, wdst):
    return pl.pallas_call(
        _proj_body,
        out_shape=[jax.ShapeDtypeStruct((N, H), jnp.int32),
                   jax.ShapeDtypeStruct((N, H), jnp.float32)],
    )(x, wsrc, wdst)


def _edgemm0_body(e_ref, w_ref, b_ref, out_ref):
    y = jnp.dot(e_ref[...], w_ref[...],
                preferred_element_type=jnp.float32) + b_ref[...]
    out_ref[...] = _pack_words(y[:, :HW], y[:, HW:])


def _edgemm0(ea, wc, be):
    kd = ea.shape[1]
    return pl.pallas_call(
        _edgemm0_body,
        grid=(E // EB,),
        in_specs=[pl.BlockSpec((EB, kd), lambda i: (i, 0)),
                  pl.BlockSpec((kd, H), lambda i: (0, 0)),
                  pl.BlockSpec((1, H), lambda i: (0, 0))],
        out_specs=pl.BlockSpec((EB, HW), lambda i: (i, 0)),
        out_shape=jax.ShapeDtypeStruct((E, HW), jnp.int32),
    )(ea, wc, be)


def _edgemmw_body(e_ref, wlo_ref, whi_ref, b_ref, out_ref):
    vlo, vhi = _unpack_words(e_ref[...])
    y = (jnp.dot(vlo, wlo_ref[...], preferred_element_type=jnp.float32)
         + jnp.dot(vhi, whi_ref[...], preferred_element_type=jnp.float32)
         + b_ref[...])
    out_ref[...] = _pack_words(y[:, :HW], y[:, HW:])


def _edgemmw(ew, wlo, whi, be):
    return pl.pallas_call(
        _edgemmw_body,
        grid=(E // EB,),
        in_specs=[pl.BlockSpec((EB, HW), lambda i: (i, 0)),
                  pl.BlockSpec((HW, H), lambda i: (0, 0)),
                  pl.BlockSpec((HW, H), lambda i: (0, 0)),
                  pl.BlockSpec((1, H), lambda i: (0, 0))],
        out_specs=pl.BlockSpec((EB, HW), lambda i: (i, 0)),
        out_shape=jax.ShapeDtypeStruct((E, HW), jnp.int32),
    )(ew, wlo, whi, be)


def _node_update(xpw, aggp, w2lo_ref, w2hi_ref, w2p_ref, b2_ref):
    xlo, xhi = _unpack_words(xpw)
    return jnp.maximum(
        jnp.dot(xlo, w2lo_ref[...], preferred_element_type=jnp.float32)
        + jnp.dot(xhi, w2hi_ref[...], preferred_element_type=jnp.float32)
        + jnp.dot(aggp, w2p_ref[...], preferred_element_type=jnp.float32)
        + b2_ref[...], 0.0)


def _postproj_body(xpw_ref, agg_ref, w2lo_ref, w2hi_ref, w2p_ref, b2_ref,
                   wsrc_ref, wdst_ref, gsrc_ref, gdst_ref, mean_ref):
    aggp = agg_ref[:N] + agg_ref[N:]
    xn = _node_update(xpw_ref[:, HW:], aggp, w2lo_ref, w2hi_ref, w2p_ref,
                      b2_ref)
    ya = jnp.dot(xn, wsrc_ref[...], preferred_element_type=jnp.float32)
    gsrc_ref[...] = jnp.concatenate(
        [_pack_words(ya[:, :HW], ya[:, HW:H]),
         _pack_words(ya[:, H:H + HW], ya[:, H + HW:])], axis=1)
    gdst_ref[...] = jnp.dot(xn, wdst_ref[...],
                            preferred_element_type=jnp.float32)
    mean_ref[...] = jnp.sum(xn, axis=0, keepdims=True) * (1.0 / N)


def _postproj(gsrc, agg, w2lo, w2hi, w2p, b2, wsrc, wdst):
    return pl.pallas_call(
        _postproj_body,
        grid=(1,),
        in_specs=[pl.BlockSpec((N, H), lambda i: (0, 0)),
                  pl.BlockSpec((2 * N, H), lambda i: (0, 0)),
                  pl.BlockSpec((HW, H), lambda i: (0, 0)),
                  pl.BlockSpec((HW, H), lambda i: (0, 0)),
                  pl.BlockSpec((H, H), lambda i: (0, 0)),
                  pl.BlockSpec((1, H), lambda i: (0, 0)),
                  pl.BlockSpec((H, 2 * H), lambda i: (0, 0)),
                  pl.BlockSpec((H, H), lambda i: (0, 0))],
        out_specs=[pl.BlockSpec((N, H), lambda i: (0, 0)),
                   pl.BlockSpec((N, H), lambda i: (0, 0)),
                   pl.BlockSpec((1, H), lambda i: (0, 0))],
        out_shape=[jax.ShapeDtypeStruct((N, H), jnp.int32),
                   jax.ShapeDtypeStruct((N, H), jnp.float32),
                   jax.ShapeDtypeStruct((1, H), jnp.float32)],
    )(gsrc, agg, w2lo, w2hi, w2p, b2, wsrc, wdst)


def _final_body(xpw_ref, agg_ref, w2lo_ref, w2hi_ref, w2p_ref, b2_ref,
                m0_ref, m1_ref, lw_ref, lb_ref, out_ref):
    aggp = agg_ref[:N] + agg_ref[N:]
    xn = _node_update(xpw_ref[:, HW:], aggp, w2lo_ref, w2hi_ref, w2p_ref,
                      b2_ref)
    m2 = jnp.sum(xn, axis=0, keepdims=True) * (1.0 / N)
    acc = (jnp.sum(m0_ref[...] * lw_ref[0, :]) +
           jnp.sum(m1_ref[...] * lw_ref[1, :]) +
           jnp.sum(m2 * lw_ref[2, :]))
    out_ref[...] = acc + lb_ref[...]


def _final(gsrc, agg, w2lo, w2hi, w2p, b2, m0, m1, lw, lb):
    return pl.pallas_call(
        _final_body,
        grid=(1,),
        in_specs=[pl.BlockSpec((N, H), lambda i: (0, 0)),
                  pl.BlockSpec((2 * N, H), lambda i: (0, 0)),
                  pl.BlockSpec((HW, H), lambda i: (0, 0)),
                  pl.BlockSpec((HW, H), lambda i: (0, 0)),
                  pl.BlockSpec((H, H), lambda i: (0, 0)),
                  pl.BlockSpec((1, H), lambda i: (0, 0)),
                  pl.BlockSpec((1, H), lambda i: (0, 0)),
                  pl.BlockSpec((1, H), lambda i: (0, 0)),
                  pl.BlockSpec((3, H), lambda i: (0, 0)),
                  pl.BlockSpec((1, 1), lambda i: (0, 0))],
        out_specs=pl.BlockSpec((1, 1), lambda i: (0, 0)),
        out_shape=jax.ShapeDtypeStruct((1, 1), jnp.float32),
    )(gsrc, agg, w2lo, w2hi, w2p, b2, m0, m1, lw, lb)


def _make_sc_edge(write_e: bool):
    mesh = plsc.VectorSubcoreMesh(core_axis_name="c", subcore_axis_name="s")
    out_type = []
    if write_e:
        out_type.append(jax.ShapeDtypeStruct((E, HW), jnp.int32))
    out_type.append(jax.ShapeDtypeStruct((NCORES * N, H), jnp.float32))
    scratch = [
        pltpu.VMEM((SCH * KCH,), jnp.int32),
        pltpu.VMEM((SCH * KCH,), jnp.int32),
        pltpu.VMEM((KCH,), jnp.int32),
        pltpu.VMEM((KCH,), jnp.int32),
        pltpu.VMEM((KCH, H), jnp.int32),
        pltpu.VMEM((KCH, H), jnp.int32),
        pltpu.VMEM((KCH, H), jnp.float32),
        pltpu.VMEM((KCH, H), jnp.float32),
        pltpu.VMEM((KCH, HW), jnp.int32),
        pltpu.VMEM((KCH, HW), jnp.int32),
        pltpu.VMEM((KCH, H), jnp.float32),
        pltpu.VMEM((KCH, H), jnp.float32),
        pltpu.VMEM_SHARED((N, H), jnp.float32),
        pltpu.SemaphoreType.DMA,
        pltpu.SemaphoreType.DMA,
        pltpu.SemaphoreType.DMA,
        pltpu.SemaphoreType.DMA,
        pltpu.SemaphoreType.DMA,
        pltpu.SemaphoreType.DMA,
        pltpu.SemaphoreType.DMA,
        pltpu.SemaphoreType.DMA,
        pltpu.SemaphoreType.DMA,
        pltpu.SemaphoreType.DMA,
        pltpu.SemaphoreType.DMA,
        pltpu.SemaphoreType.DMA,
    ]

    def body(gsrc, gdst, cterm, srci, dsti, zeros, *rest):
        if write_e:
            e_hbm, agg_hbm = rest[0], rest[1]
            rest = rest[2:]
        else:
            agg_hbm = rest[0]
            rest = rest[1:]
        (sisc, disc, dx0, dx1, g0, g1, d0, d1, c0, c1, mv0, mv1, aggsh,
         sg0, sg1, sd0, sd1, sc0, sc1, se0, se1, sx0, sx1, ss0, ss1) = rest
        cid = lax.axis_index("c")
        sid = lax.axis_index("s")
        wid = sid * NCORES + cid
        r0 = sid * RPT

        pltpu.sync_copy(zeros.at[pl.ds(r0, RPT)], aggsh.at[pl.ds(r0, RPT)])

        @pl.when(sid == NSUB - 1)
        def _():
            pltpu.sync_copy(zeros.at[pl.ds(TAIL0, TAILN)],
                            aggsh.at[pl.ds(TAIL0, TAILN)])

        plsc.subcore_barrier()

        slots = ((g0, d0, c0, mv0, dx0, sg0, sd0, sc0, se0, sx0, ss0),
                 (g1, d1, c1, mv1, dx1, sg1, sd1, sc1, se1, sx1, ss1))
        ebase0 = wid * EPW

        def issue(p, s, t):
            g_, d_, c_, mv_, dx_, sg_, sd_, sc_, se_, sx_, ss_ = slots[p]
            base = ebase0 + (s * SCH + t) * KCH
            tsl = pl.ds(t * KCH, KCH)

            def _drain():
                if write_e:
                    pltpu.make_async_copy(c_, e_hbm.at[pl.ds(0, KCH)],
                                          se_).wait()
                pltpu.make_async_copy(mv_, aggsh.at[dx_], ss_).wait()

            if t >= 2:
                _drain()
            else:
                @pl.when(s > 0)
                def _():
                    _drain()
            pltpu.async_copy(gsrc.at[sisc.at[tsl]], g_, sg_)
            pltpu.async_copy(gdst.at[disc.at[tsl]], d_, sd_)
            pltpu.async_copy(cterm.at[pl.ds(base, KCH)], c_, sc_)
            pltpu.async_copy(dsti.at[pl.ds(base, KCH)], dx_, sx_)

        def process(p, s, t):
            g_, d_, c_, mv_, dx_, sg_, sd_, sc_, se_, sx_, ss_ = slots[p]
            base = ebase0 + (s * SCH + t) * KCH
            tsl = pl.ds(t * KCH, KCH)
            pltpu.make_async_copy(gsrc.at[sisc.at[tsl]], g_, sg_).wait()
            pltpu.make_async_copy(gdst.at[disc.at[tsl]], d_, sd_).wait()
            pltpu.make_async_copy(cterm.at[pl.ds(base, KCH)], c_, sc_).wait()

            @plsc.parallel_loop(0, KCH, 1, unroll=2)
            def row(j):
                for qq in range(4):
                    slw = pl.ds(16 * qq, 16)
                    a0, a1 = _unpack_words(g_[j, slw])
                    b0 = d_[j, pl.ds(16 * qq, 16)]
                    b1 = d_[j, pl.ds(HW + 16 * qq, 16)]
                    k0, k1 = _unpack_words(c_[j, slw])
                    e0 = jnp.maximum(a0 + b0 + k0, 0.0)
                    e1 = jnp.maximum(a1 + b1 + k1, 0.0)
                    x0, x1 = _unpack_words(g_[j, pl.ds(HW + 16 * qq, 16)])
                    m0 = jnp.maximum(x0 + e0, 0.0)
                    m1 = jnp.maximum(x1 + e1, 0.0)
                    if write_e:
                        c_[j, slw] = _pack_words(e0, e1)
                    mv_[j, pl.ds(32 * qq, 16)] = m0
                    mv_[j, pl.ds(32 * qq + 16, 16)] = m1

            if write_e:
                pltpu.async_copy(c_, e_hbm.at[pl.ds(base, KCH)], se_)
            pltpu.make_async_copy(dsti.at[pl.ds(base, KCH)], dx_, sx_).wait()
            pltpu.async_copy(mv_, aggsh.at[dx_], ss_, add=True)

        def super_chunk(s, carry):
            sbase = ebase0 + s * (SCH * KCH)
            pltpu.sync_copy(srci.at[pl.ds(sbase, SCH * KCH)], sisc)
            pltpu.sync_copy(dsti.at[pl.ds(sbase, SCH * KCH)], disc)
            issue(0, s, 0)
            for t in range(SCH):
                if t + 1 < SCH:
                    issue((t + 1) % 2, s, t + 1)
                process(t % 2, s, t)
            return carry

        lax.fori_loop(0, NCHUNK // SCH, super_chunk, 0)
        if write_e:
            pltpu.make_async_copy(c0, e_hbm.at[pl.ds(0, KCH)], se0).wait()
            pltpu.make_async_copy(c1, e_hbm.at[pl.ds(0, KCH)], se1).wait()
        pltpu.make_async_copy(mv0, aggsh.at[dx0], ss0).wait()
        pltpu.make_async_copy(mv1, aggsh.at[dx1], ss1).wait()
        plsc.subcore_barrier()

        o0 = cid * N
        pltpu.sync_copy(aggsh.at[pl.ds(r0, RPT)], agg_hbm.at[pl.ds(o0 + r0, RPT)])

        @pl.when(sid == NSUB - 1)
        def _():
            pltpu.sync_copy(aggsh.at[pl.ds(TAIL0, TAILN)],
                            agg_hbm.at[pl.ds(o0 + TAIL0, TAILN)])

    return pl.kernel(body, out_type=out_type, mesh=mesh, scratch_types=scratch)


_sc_edge_we = _make_sc_edge(True)
_sc_edge_noe = _make_sc_edge(False)


def kernel(x, edge_index, edge_attr, batch, We0, be0, Wx0, W20, b20,
           We1, be1, Wx1, W21, b21, We2, be2, Wx2, W22, b22, lin_W, lin_b):
    src = edge_index[0]
    dst = edge_index[1]
    zeros = jnp.zeros((N, H), jnp.float32)
    params = [(We0, be0, Wx0, W20, b20), (We1, be1, Wx1, W21, b21),
              (We2, be2, Wx2, W22, b22)]

    tau = jnp.array([16 * (p // 32) + (p % 32) if p % 32 < 16
                     else HW + 16 * (p // 32) + (p % 32 - 16)
                     for p in range(H)], dtype=jnp.int32)

    wsplit = []
    for (We, be, Wx, W2, b2) in params:
        Wa = We[:H]
        Wb = We[H:2 * H]
        Wc = We[2 * H:]
        wsrc = jnp.concatenate([Wa, Wx], axis=1)
        wsplit.append((wsrc, Wb, Wc, be.reshape(1, H),
                       W2[:HW], W2[HW:], W2[tau, :], b2.reshape(1, H)))

    ea = edge_attr
    means = []
    gsrc, gdst = _proj(x, wsplit[0][0], wsplit[0][1])
    for b in range(2):
        wsrc, Wb, Wc, be1h, w2lo, w2hi, w2p, b21h = wsplit[b]
        if b == 0:
            C = _edgemm0(ea, Wc, be1h)
        else:
            C = _edgemmw(ea, Wc[:HW], Wc[HW:], be1h)
        e_new, agg = _sc_edge_we(gsrc, gdst, C, src, dst, zeros)
        gsrc, gdst, mean = _postproj(gsrc, agg, w2lo, w2hi, w2p, b21h,
                                     wsplit[b + 1][0], wsplit[b + 1][1])
        ea = e_new
        means.append(mean)

    wsrc, Wb, Wc, be1h, w2lo, w2hi, w2p, b21h = wsplit[2]
    C = _edgemmw(ea, Wc[:HW], Wc[HW:], be1h)
    (agg,) = _sc_edge_noe(gsrc, gdst, C, src, dst, zeros)
    lw = lin_W.reshape(3, H)
    return _final(gsrc, agg, w2lo, w2hi, w2p, b21h, means[0], means[1], lw,
                  lin_b.reshape(1, 1))


# speedup vs baseline: 5.5276x; 1.0037x over previous
import functools

import jax
import jax.numpy as jnp
from jax import lax
from jax.experimental import pallas as pl
from jax.experimental.pallas import tpu as pltpu
from jax.experimental.pallas import tpu_sc as plsc

N = 10000
E = 320000
H = 128
HW = H // 2
NCORES = 2
NSUB = 16
NW = NCORES * NSUB
EPW = E // NW
KCH = 40
NCHUNK = EPW // KCH
RPT = 624
TAIL0 = RPT * NSUB
TAILN = N - TAIL0
EB = 8000
SCH = 10


def _rne_hi(x):
    bits = lax.bitcast_convert_type(x, jnp.uint32)
    t = bits + jnp.uint32(0x7FFF) + ((bits >> 16) & jnp.uint32(1))
    return t & jnp.uint32(0xFFFF0000)


def _pack_words(lo, hi):
    return lax.bitcast_convert_type(_rne_hi(hi) | (_rne_hi(lo) >> 16),
                                    jnp.int32)


def _unpack_words(w):
    wu = lax.bitcast_convert_type(w, jnp.uint32)
    vlo = lax.bitcast_convert_type(wu << 16, jnp.float32)
    vhi = lax.bitcast_convert_type(wu & jnp.uint32(0xFFFF0000), jnp.float32)
    return vlo, vhi


def _pack_words_fast(lo, hi):
    bl = lax.bitcast_convert_type(lo, jnp.uint32) + jnp.uint32(0x8000)
    bh = lax.bitcast_convert_type(hi, jnp.uint32) + jnp.uint32(0x8000)
    return lax.bitcast_convert_type(
        (bh & jnp.uint32(0xFFFF0000)) | (bl >> 16), jnp.int32)


def _proj_body(x_ref, wsrc_ref, wdst_ref, gsrc_ref, gdst_ref):
    x = x_ref[...]
    ya = jnp.dot(x, wsrc_ref[...], preferred_element_type=jnp.float32)
    gsrc_ref[...] = jnp.concatenate(
        [_pack_words(ya[:, :HW], ya[:, HW:H]),
         _pack_words(ya[:, H:H + HW], ya[:, H + HW:])], axis=1)
    gdst_ref[...] = jnp.dot(x, wdst_ref[...],
                            preferred_element_type=jnp.float32)


def _proj(x, wsrc, wdst):
    return pl.pallas_call(
        _proj_body,
        out_shape=[jax.ShapeDtypeStruct((N, H), jnp.int32),
                   jax.ShapeDtypeStruct((N, H), jnp.float32)],
    )(x, wsrc, wdst)


def _edgemm0_body(e_ref, w_ref, b_ref, out_ref):
    y = jnp.dot(e_ref[...], w_ref[...],
                preferred_element_type=jnp.float32) + b_ref[...]
    out_ref[...] = _pack_words(y[:, :HW], y[:, HW:])


def _edgemm0(ea, wc, be):
    kd = ea.shape[1]
    return pl.pallas_call(
        _edgemm0_body,
        grid=(E // EB,),
        in_specs=[pl.BlockSpec((EB, kd), lambda i: (i, 0)),
                  pl.BlockSpec((kd, H), lambda i: (0, 0)),
                  pl.BlockSpec((1, H), lambda i: (0, 0))],
        out_specs=pl.BlockSpec((EB, HW), lambda i: (i, 0)),
        out_shape=jax.ShapeDtypeStruct((E, HW), jnp.int32),
    )(ea, wc, be)


def _edgemmw_body(e_ref, wlo_ref, whi_ref, b_ref, out_ref):
    vlo, vhi = _unpack_words(e_ref[...])
    y = (jnp.dot(vlo, wlo_ref[...], preferred_element_type=jnp.float32)
         + jnp.dot(vhi, whi_ref[...], preferred_element_type=jnp.float32)
         + b_ref[...])
    out_ref[...] = _pack_words(y[:, :HW], y[:, HW:])


def _edgemmw(ew, wlo, whi, be):
    return pl.pallas_call(
        _edgemmw_body,
        grid=(E // EB,),
        in_specs=[pl.BlockSpec((EB, HW), lambda i: (i, 0)),
                  pl.BlockSpec((HW, H), lambda i: (0, 0)),
                  pl.BlockSpec((HW, H), lambda i: (0, 0)),
                  pl.BlockSpec((1, H), lambda i: (0, 0))],
        out_specs=pl.BlockSpec((EB, HW), lambda i: (i, 0)),
        out_shape=jax.ShapeDtypeStruct((E, HW), jnp.int32),
    )(ew, wlo, whi, be)


def _node_update(xpw, aggp, w2lo_ref, w2hi_ref, w2p_ref, b2_ref):
    xlo, xhi = _unpack_words(xpw)
    return jnp.maximum(
        jnp.dot(xlo, w2lo_ref[...], preferred_element_type=jnp.float32)
        + jnp.dot(xhi, w2hi_ref[...], preferred_element_type=jnp.float32)
        + jnp.dot(aggp, w2p_ref[...], preferred_element_type=jnp.float32)
        + b2_ref[...], 0.0)


def _postproj_body(xpw_ref, agg_ref, w2lo_ref, w2hi_ref, w2p_ref, b2_ref,
                   wsrc_ref, wdst_ref, gsrc_ref, gdst_ref, mean_ref):
    aggp = agg_ref[:N] + agg_ref[N:]
    xn = _node_update(xpw_ref[:, HW:], aggp, w2lo_ref, w2hi_ref, w2p_ref,
                      b2_ref)
    ya = jnp.dot(xn, wsrc_ref[...], preferred_element_type=jnp.float32)
    gsrc_ref[...] = jnp.concatenate(
        [_pack_words(ya[:, :HW], ya[:, HW:H]),
         _pack_words(ya[:, H:H + HW], ya[:, H + HW:])], axis=1)
    gdst_ref[...] = jnp.dot(xn, wdst_ref[...],
                            preferred_element_type=jnp.float32)
    mean_ref[...] = jnp.sum(xn, axis=0, keepdims=True) * (1.0 / N)


def _postproj(gsrc, agg, w2lo, w2hi, w2p, b2, wsrc, wdst):
    return pl.pallas_call(
        _postproj_body,
        grid=(1,),
        in_specs=[pl.BlockSpec((N, H), lambda i: (0, 0)),
                  pl.BlockSpec((2 * N, H), lambda i: (0, 0)),
                  pl.BlockSpec((HW, H), lambda i: (0, 0)),
                  pl.BlockSpec((HW, H), lambda i: (0, 0)),
                  pl.BlockSpec((H, H), lambda i: (0, 0)),
                  pl.BlockSpec((1, H), lambda i: (0, 0)),
                  pl.BlockSpec((H, 2 * H), lambda i: (0, 0)),
                  pl.BlockSpec((H, H), lambda i: (0, 0))],
        out_specs=[pl.BlockSpec((N, H), lambda i: (0, 0)),
                   pl.BlockSpec((N, H), lambda i: (0, 0)),
                   pl.BlockSpec((1, H), lambda i: (0, 0))],
        out_shape=[jax.ShapeDtypeStruct((N, H), jnp.int32),
                   jax.ShapeDtypeStruct((N, H), jnp.float32),
                   jax.ShapeDtypeStruct((1, H), jnp.float32)],
    )(gsrc, agg, w2lo, w2hi, w2p, b2, wsrc, wdst)


def _final_body(xpw_ref, agg_ref, w2lo_ref, w2hi_ref, w2p_ref, b2_ref,
                m0_ref, m1_ref, lw_ref, lb_ref, out_ref):
    aggp = agg_ref[:N] + agg_ref[N:]
    xn = _node_update(xpw_ref[:, HW:], aggp, w2lo_ref, w2hi_ref, w2p_ref,
                      b2_ref)
    m2 = jnp.sum(xn, axis=0, keepdims=True) * (1.0 / N)
    acc = (jnp.sum(m0_ref[...] * lw_ref[0, :]) +
           jnp.sum(m1_ref[...] * lw_ref[1, :]) +
           jnp.sum(m2 * lw_ref[2, :]))
    out_ref[...] = acc + lb_ref[...]


def _final(gsrc, agg, w2lo, w2hi, w2p, b2, m0, m1, lw, lb):
    return pl.pallas_call(
        _final_body,
        grid=(1,),
        in_specs=[pl.BlockSpec((N, H), lambda i: (0, 0)),
                  pl.BlockSpec((2 * N, H), lambda i: (0, 0)),
                  pl.BlockSpec((HW, H), lambda i: (0, 0)),
                  pl.BlockSpec((HW, H), lambda i: (0, 0)),
                  pl.BlockSpec((H, H), lambda i: (0, 0)),
                  pl.BlockSpec((1, H), lambda i: (0, 0)),
                  pl.BlockSpec((1, H), lambda i: (0, 0)),
                  pl.BlockSpec((1, H), lambda i: (0, 0)),
                  pl.BlockSpec((3, H), lambda i: (0, 0)),
                  pl.BlockSpec((1, 1), lambda i: (0, 0))],
        out_specs=pl.BlockSpec((1, 1), lambda i: (0, 0)),
        out_shape=jax.ShapeDtypeStruct((1, 1), jnp.float32),
    )(gsrc, agg, w2lo, w2hi, w2p, b2, m0, m1, lw, lb)


def _make_sc_edge(write_e: bool):
    mesh = plsc.VectorSubcoreMesh(core_axis_name="c", subcore_axis_name="s")
    out_type = []
    if write_e:
        out_type.append(jax.ShapeDtypeStruct((E, HW), jnp.int32))
    out_type.append(jax.ShapeDtypeStruct((NCORES * N, H), jnp.float32))
    scratch = [
        pltpu.VMEM((SCH * KCH,), jnp.int32),
        pltpu.VMEM((SCH * KCH,), jnp.int32),
        pltpu.VMEM((KCH,), jnp.int32),
        pltpu.VMEM((KCH,), jnp.int32),
        pltpu.VMEM((KCH, H), jnp.int32),
        pltpu.VMEM((KCH, H), jnp.int32),
        pltpu.VMEM((KCH, H), jnp.float32),
        pltpu.VMEM((KCH, H), jnp.float32),
        pltpu.VMEM((KCH, HW), jnp.int32),
        pltpu.VMEM((KCH, HW), jnp.int32),
        pltpu.VMEM((KCH, H), jnp.float32),
        pltpu.VMEM((KCH, H), jnp.float32),
        pltpu.VMEM_SHARED((N, H), jnp.float32),
        pltpu.SemaphoreType.DMA,
        pltpu.SemaphoreType.DMA,
        pltpu.SemaphoreType.DMA,
        pltpu.SemaphoreType.DMA,
        pltpu.SemaphoreType.DMA,
        pltpu.SemaphoreType.DMA,
        pltpu.SemaphoreType.DMA,
        pltpu.SemaphoreType.DMA,
        pltpu.SemaphoreType.DMA,
        pltpu.SemaphoreType.DMA,
        pltpu.SemaphoreType.DMA,
        pltpu.SemaphoreType.DMA,
    ]

    def body(gsrc, gdst, cterm, srci, dsti, zeros, *rest):
        if write_e:
            e_hbm, agg_hbm = rest[0], rest[1]
            rest = rest[2:]
        else:
            agg_hbm = rest[0]
            rest = rest[1:]
        (sisc, disc, dx0, dx1, g0, g1, d0, d1, c0, c1, mv0, mv1, aggsh,
         sg0, sg1, sd0, sd1, sc0, sc1, se0, se1, sx0, sx1, ss0, ss1) = rest
        cid = lax.axis_index("c")
        sid = lax.axis_index("s")
        wid = sid * NCORES + cid
        r0 = sid * RPT

        pltpu.sync_copy(zeros.at[pl.ds(r0, RPT)], aggsh.at[pl.ds(r0, RPT)])

        @pl.when(sid == NSUB - 1)
        def _():
            pltpu.sync_copy(zeros.at[pl.ds(TAIL0, TAILN)],
                            aggsh.at[pl.ds(TAIL0, TAILN)])

        plsc.subcore_barrier()

        slots = ((g0, d0, c0, mv0, dx0, sg0, sd0, sc0, se0, sx0, ss0),
                 (g1, d1, c1, mv1, dx1, sg1, sd1, sc1, se1, sx1, ss1))
        ebase0 = wid * EPW

        def issue(p, s, t):
            g_, d_, c_, mv_, dx_, sg_, sd_, sc_, se_, sx_, ss_ = slots[p]
            base = ebase0 + (s * SCH + t) * KCH
            tsl = pl.ds(t * KCH, KCH)

            def _drain():
                if write_e:
                    pltpu.make_async_copy(c_, e_hbm.at[pl.ds(0, KCH)],
                                          se_).wait()
                pltpu.make_async_copy(mv_, aggsh.at[dx_], ss_).wait()

            if t >= 2:
                _drain()
            else:
                @pl.when(s > 0)
                def _():
                    _drain()
            pltpu.async_copy(gsrc.at[sisc.at[tsl]], g_, sg_)
            pltpu.async_copy(gdst.at[disc.at[tsl]], d_, sd_)
            pltpu.async_copy(cterm.at[pl.ds(base, KCH)], c_, sc_)
            pltpu.async_copy(dsti.at[pl.ds(base, KCH)], dx_, sx_)

        def process(p, s, t):
            g_, d_, c_, mv_, dx_, sg_, sd_, sc_, se_, sx_, ss_ = slots[p]
            base = ebase0 + (s * SCH + t) * KCH
            tsl = pl.ds(t * KCH, KCH)
            pltpu.make_async_copy(gsrc.at[sisc.at[tsl]], g_, sg_).wait()
            pltpu.make_async_copy(gdst.at[disc.at[tsl]], d_, sd_).wait()
            pltpu.make_async_copy(cterm.at[pl.ds(base, KCH)], c_, sc_).wait()

            @plsc.parallel_loop(0, KCH, 1, unroll=4)
            def row(j):
                for qq in range(4):
                    slw = pl.ds(16 * qq, 16)
                    a0, a1 = _unpack_words(g_[j, slw])
                    b0 = d_[j, pl.ds(16 * qq, 16)]
                    b1 = d_[j, pl.ds(HW + 16 * qq, 16)]
                    k0, k1 = _unpack_words(c_[j, slw])
                    e0 = jnp.maximum(a0 + b0 + k0, 0.0)
                    e1 = jnp.maximum(a1 + b1 + k1, 0.0)
                    x0, x1 = _unpack_words(g_[j, pl.ds(HW + 16 * qq, 16)])
                    m0 = jnp.maximum(x0 + e0, 0.0)
                    m1 = jnp.maximum(x1 + e1, 0.0)
                    if write_e:
                        c_[j, slw] = _pack_words_fast(e0, e1)
                    mv_[j, pl.ds(32 * qq, 16)] = m0
                    mv_[j, pl.ds(32 * qq + 16, 16)] = m1

            if write_e:
                pltpu.async_copy(c_, e_hbm.at[pl.ds(base, KCH)], se_)
            pltpu.make_async_copy(dsti.at[pl.ds(base, KCH)], dx_, sx_).wait()
            pltpu.async_copy(mv_, aggsh.at[dx_], ss_, add=True)

        def super_chunk(s, carry):
            sbase = ebase0 + s * (SCH * KCH)
            pltpu.sync_copy(srci.at[pl.ds(sbase, SCH * KCH)], sisc)
            pltpu.sync_copy(dsti.at[pl.ds(sbase, SCH * KCH)], disc)
            issue(0, s, 0)
            for t in range(SCH):
                if t + 1 < SCH:
                    issue((t + 1) % 2, s, t + 1)
                process(t % 2, s, t)
            return carry

        lax.fori_loop(0, NCHUNK // SCH, super_chunk, 0)
        if write_e:
            pltpu.make_async_copy(c0, e_hbm.at[pl.ds(0, KCH)], se0).wait()
            pltpu.make_async_copy(c1, e_hbm.at[pl.ds(0, KCH)], se1).wait()
        pltpu.make_async_copy(mv0, aggsh.at[dx0], ss0).wait()
        pltpu.make_async_copy(mv1, aggsh.at[dx1], ss1).wait()
        plsc.subcore_barrier()

        o0 = cid * N
        pltpu.sync_copy(aggsh.at[pl.ds(r0, RPT)], agg_hbm.at[pl.ds(o0 + r0, RPT)])

        @pl.when(sid == NSUB - 1)
        def _():
            pltpu.sync_copy(aggsh.at[pl.ds(TAIL0, TAILN)],
                            agg_hbm.at[pl.ds(o0 + TAIL0, TAILN)])

    return pl.kernel(body, out_type=out_type, mesh=mesh, scratch_types=scratch)


_sc_edge_we = _make_sc_edge(True)
_sc_edge_noe = _make_sc_edge(False)


def kernel(x, edge_index, edge_attr, batch, We0, be0, Wx0, W20, b20,
           We1, be1, Wx1, W21, b21, We2, be2, Wx2, W22, b22, lin_W, lin_b):
    src = edge_index[0]
    dst = edge_index[1]
    zeros = jnp.zeros((N, H), jnp.float32)
    params = [(We0, be0, Wx0, W20, b20), (We1, be1, Wx1, W21, b21),
              (We2, be2, Wx2, W22, b22)]

    tau = jnp.array([16 * (p // 32) + (p % 32) if p % 32 < 16
                     else HW + 16 * (p // 32) + (p % 32 - 16)
                     for p in range(H)], dtype=jnp.int32)

    wsplit = []
    for (We, be, Wx, W2, b2) in params:
        Wa = We[:H]
        Wb = We[H:2 * H]
        Wc = We[2 * H:]
        wsrc = jnp.concatenate([Wa, Wx], axis=1)
        wsplit.append((wsrc, Wb, Wc, be.reshape(1, H),
                       W2[:HW], W2[HW:], W2[tau, :], b2.reshape(1, H)))

    ea = edge_attr
    means = []
    gsrc, gdst = _proj(x, wsplit[0][0], wsplit[0][1])
    for b in range(2):
        wsrc, Wb, Wc, be1h, w2lo, w2hi, w2p, b21h = wsplit[b]
        if b == 0:
            C = _edgemm0(ea, Wc, be1h)
        else:
            C = _edgemmw(ea, Wc[:HW], Wc[HW:], be1h)
        e_new, agg = _sc_edge_we(gsrc, gdst, C, src, dst, zeros)
        gsrc, gdst, mean = _postproj(gsrc, agg, w2lo, w2hi, w2p, b21h,
                                     wsplit[b + 1][0], wsplit[b + 1][1])
        ea = e_new
        means.append(mean)

    wsrc, Wb, Wc, be1h, w2lo, w2hi, w2p, b21h = wsplit[2]
    C = _edgemmw(ea, Wc[:HW], Wc[HW:], be1h)
    (agg,) = _sc_edge_noe(gsrc, gdst, C, src, dst, zeros)
    lw = lin_W.reshape(3, H)
    return _final(gsrc, agg, w2lo, w2hi, w2p, b21h, means[0], means[1], lw,
                  lin_b.reshape(1, 1))


# speedup vs baseline: 5.6319x; 1.0189x over previous
import functools

import jax
import jax.numpy as jnp
from jax import lax
from jax.experimental import pallas as pl
from jax.experimental.pallas import tpu as pltpu
from jax.experimental.pallas import tpu_sc as plsc

N = 10000
E = 320000
H = 128
HW = H // 2
NCORES = 2
NSUB = 16
NW = NCORES * NSUB
EPW = E // NW
KCH = 40
NCHUNK = EPW // KCH
RPT = 624
TAIL0 = RPT * NSUB
TAILN = N - TAIL0
EB = 8000
SCH = 10


def _rne_hi(x):
    bits = lax.bitcast_convert_type(x, jnp.uint32)
    t = bits + jnp.uint32(0x7FFF) + ((bits >> 16) & jnp.uint32(1))
    return t & jnp.uint32(0xFFFF0000)


def _pack_words(lo, hi):
    return lax.bitcast_convert_type(_rne_hi(hi) | (_rne_hi(lo) >> 16),
                                    jnp.int32)


def _unpack_words(w):
    wu = lax.bitcast_convert_type(w, jnp.uint32)
    vlo = lax.bitcast_convert_type(wu << 16, jnp.float32)
    vhi = lax.bitcast_convert_type(wu & jnp.uint32(0xFFFF0000), jnp.float32)
    return vlo, vhi


def _pack_words_fast(lo, hi):
    bl = lax.bitcast_convert_type(lo, jnp.uint32) + jnp.uint32(0x8000)
    bh = lax.bitcast_convert_type(hi, jnp.uint32) + jnp.uint32(0x8000)
    return lax.bitcast_convert_type(
        (bh & jnp.uint32(0xFFFF0000)) | (bl >> 16), jnp.int32)


def _proj_body(x_ref, wsrc_ref, wdst_ref, gsrc_ref, gdst_ref):
    x = x_ref[...]
    ya = jnp.dot(x, wsrc_ref[...], preferred_element_type=jnp.float32)
    gsrc_ref[...] = jnp.concatenate(
        [_pack_words(ya[:, :HW], ya[:, HW:H]),
         _pack_words(ya[:, H:H + HW], ya[:, H + HW:])], axis=1)
    gdst_ref[...] = jnp.dot(x, wdst_ref[...],
                            preferred_element_type=jnp.float32)


def _proj(x, wsrc, wdst):
    return pl.pallas_call(
        _proj_body,
        out_shape=[jax.ShapeDtypeStruct((N, H), jnp.int32),
                   jax.ShapeDtypeStruct((N, H), jnp.float32)],
    )(x, wsrc, wdst)


def _edgemm0_body(e_ref, w_ref, b_ref, out_ref):
    y = jnp.dot(e_ref[...], w_ref[...],
                preferred_element_type=jnp.float32) + b_ref[...]
    out_ref[...] = _pack_words(y[:, :HW], y[:, HW:])


def _edgemm0(ea, wc, be):
    kd = ea.shape[1]
    return pl.pallas_call(
        _edgemm0_body,
        grid=(E // EB,),
        in_specs=[pl.BlockSpec((EB, kd), lambda i: (i, 0)),
                  pl.BlockSpec((kd, H), lambda i: (0, 0)),
                  pl.BlockSpec((1, H), lambda i: (0, 0))],
        out_specs=pl.BlockSpec((EB, HW), lambda i: (i, 0)),
        out_shape=jax.ShapeDtypeStruct((E, HW), jnp.int32),
    )(ea, wc, be)


def _edgemmw_body(e_ref, wlo_ref, whi_ref, b_ref, out_ref):
    vlo, vhi = _unpack_words(e_ref[...])
    y = (jnp.dot(vlo, wlo_ref[...], preferred_element_type=jnp.float32)
         + jnp.dot(vhi, whi_ref[...], preferred_element_type=jnp.float32)
         + b_ref[...])
    out_ref[...] = _pack_words(y[:, :HW], y[:, HW:])


def _edgemmw(ew, wlo, whi, be):
    return pl.pallas_call(
        _edgemmw_body,
        grid=(E // EB,),
        in_specs=[pl.BlockSpec((EB, HW), lambda i: (i, 0)),
                  pl.BlockSpec((HW, H), lambda i: (0, 0)),
                  pl.BlockSpec((HW, H), lambda i: (0, 0)),
                  pl.BlockSpec((1, H), lambda i: (0, 0))],
        out_specs=pl.BlockSpec((EB, HW), lambda i: (i, 0)),
        out_shape=jax.ShapeDtypeStruct((E, HW), jnp.int32),
    )(ew, wlo, whi, be)


def _node_update(xpw, aggp, w2lo_ref, w2hi_ref, w2p_ref, b2_ref):
    xlo, xhi = _unpack_words(xpw)
    return jnp.maximum(
        jnp.dot(xlo, w2lo_ref[...], preferred_element_type=jnp.float32)
        + jnp.dot(xhi, w2hi_ref[...], preferred_element_type=jnp.float32)
        + jnp.dot(aggp, w2p_ref[...], preferred_element_type=jnp.float32)
        + b2_ref[...], 0.0)


def _postproj_body(xpw_ref, agg_ref, w2lo_ref, w2hi_ref, w2p_ref, b2_ref,
                   wsrc_ref, wdst_ref, gsrc_ref, gdst_ref, mean_ref):
    aggp = agg_ref[:N] + agg_ref[N:]
    xn = _node_update(xpw_ref[:, HW:], aggp, w2lo_ref, w2hi_ref, w2p_ref,
                      b2_ref)
    ya = jnp.dot(xn, wsrc_ref[...], preferred_element_type=jnp.float32)
    gsrc_ref[...] = jnp.concatenate(
        [_pack_words(ya[:, :HW], ya[:, HW:H]),
         _pack_words(ya[:, H:H + HW], ya[:, H + HW:])], axis=1)
    gdst_ref[...] = jnp.dot(xn, wdst_ref[...],
                            preferred_element_type=jnp.float32)
    mean_ref[...] = jnp.sum(xn, axis=0, keepdims=True) * (1.0 / N)


def _postproj(gsrc, agg, w2lo, w2hi, w2p, b2, wsrc, wdst):
    return pl.pallas_call(
        _postproj_body,
        grid=(1,),
        in_specs=[pl.BlockSpec((N, H), lambda i: (0, 0)),
                  pl.BlockSpec((2 * N, H), lambda i: (0, 0)),
                  pl.BlockSpec((HW, H), lambda i: (0, 0)),
                  pl.BlockSpec((HW, H), lambda i: (0, 0)),
                  pl.BlockSpec((H, H), lambda i: (0, 0)),
                  pl.BlockSpec((1, H), lambda i: (0, 0)),
                  pl.BlockSpec((H, 2 * H), lambda i: (0, 0)),
                  pl.BlockSpec((H, H), lambda i: (0, 0))],
        out_specs=[pl.BlockSpec((N, H), lambda i: (0, 0)),
                   pl.BlockSpec((N, H), lambda i: (0, 0)),
                   pl.BlockSpec((1, H), lambda i: (0, 0))],
        out_shape=[jax.ShapeDtypeStruct((N, H), jnp.int32),
                   jax.ShapeDtypeStruct((N, H), jnp.float32),
                   jax.ShapeDtypeStruct((1, H), jnp.float32)],
    )(gsrc, agg, w2lo, w2hi, w2p, b2, wsrc, wdst)


def _final_body(xpw_ref, agg_ref, w2lo_ref, w2hi_ref, w2p_ref, b2_ref,
                m0_ref, m1_ref, lw_ref, lb_ref, out_ref):
    aggp = agg_ref[:N] + agg_ref[N:]
    xn = _node_update(xpw_ref[:, HW:], aggp, w2lo_ref, w2hi_ref, w2p_ref,
                      b2_ref)
    m2 = jnp.sum(xn, axis=0, keepdims=True) * (1.0 / N)
    acc = (jnp.sum(m0_ref[...] * lw_ref[0, :]) +
           jnp.sum(m1_ref[...] * lw_ref[1, :]) +
           jnp.sum(m2 * lw_ref[2, :]))
    out_ref[...] = acc + lb_ref[...]


def _final(gsrc, agg, w2lo, w2hi, w2p, b2, m0, m1, lw, lb):
    return pl.pallas_call(
        _final_body,
        grid=(1,),
        in_specs=[pl.BlockSpec((N, H), lambda i: (0, 0)),
                  pl.BlockSpec((2 * N, H), lambda i: (0, 0)),
                  pl.BlockSpec((HW, H), lambda i: (0, 0)),
                  pl.BlockSpec((HW, H), lambda i: (0, 0)),
                  pl.BlockSpec((H, H), lambda i: (0, 0)),
                  pl.BlockSpec((1, H), lambda i: (0, 0)),
                  pl.BlockSpec((1, H), lambda i: (0, 0)),
                  pl.BlockSpec((1, H), lambda i: (0, 0)),
                  pl.BlockSpec((3, H), lambda i: (0, 0)),
                  pl.BlockSpec((1, 1), lambda i: (0, 0))],
        out_specs=pl.BlockSpec((1, 1), lambda i: (0, 0)),
        out_shape=jax.ShapeDtypeStruct((1, 1), jnp.float32),
    )(gsrc, agg, w2lo, w2hi, w2p, b2, m0, m1, lw, lb)


def _make_sc_edge(write_e: bool):
    mesh = plsc.VectorSubcoreMesh(core_axis_name="c", subcore_axis_name="s")
    out_type = []
    if write_e:
        out_type.append(jax.ShapeDtypeStruct((E, HW), jnp.int32))
    out_type.append(jax.ShapeDtypeStruct((NCORES * N, H), jnp.float32))
    scratch = [
        pltpu.VMEM((SCH * KCH,), jnp.int32),
        pltpu.VMEM((SCH * KCH,), jnp.int32),
        pltpu.VMEM((KCH,), jnp.int32),
        pltpu.VMEM((KCH,), jnp.int32),
        pltpu.VMEM((KCH, H), jnp.int32),
        pltpu.VMEM((KCH, H), jnp.int32),
        pltpu.VMEM((KCH, H), jnp.int32),
        pltpu.VMEM((KCH, H), jnp.float32),
        pltpu.VMEM((KCH, H), jnp.float32),
        pltpu.VMEM((KCH, HW), jnp.int32),
        pltpu.VMEM((KCH, HW), jnp.int32),
        pltpu.VMEM((KCH, H), jnp.float32),
        pltpu.VMEM((KCH, H), jnp.float32),
        pltpu.VMEM_SHARED((N, H), jnp.float32),
    ] + [pltpu.SemaphoreType.DMA] * 13

    def body(gsrc, gdst, cterm, srci, dsti, zeros, *rest):
        if write_e:
            e_hbm, agg_hbm = rest[0], rest[1]
            rest = rest[2:]
        else:
            agg_hbm = rest[0]
            rest = rest[1:]
        (sisc, disc, dx0, dx1, g0, g1, g2, d0, d1, c0, c1,
         mv0, mv1, aggsh, sg0, sg1, sg2, sd0, sd1, sc0, sc1,
         se0, se1, sx0, sx1, ss0, ss1) = rest
        cid = lax.axis_index("c")
        sid = lax.axis_index("s")
        wid = sid * NCORES + cid
        r0 = sid * RPT

        pltpu.sync_copy(zeros.at[pl.ds(r0, RPT)], aggsh.at[pl.ds(r0, RPT)])

        @pl.when(sid == NSUB - 1)
        def _():
            pltpu.sync_copy(zeros.at[pl.ds(TAIL0, TAILN)],
                            aggsh.at[pl.ds(TAIL0, TAILN)])

        plsc.subcore_barrier()

        gl = ((g0, sg0), (g1, sg1), (g2, sg2))
        dc = ((d0, c0, sd0, sc0, se0), (d1, c1, sd1, sc1, se1))
        st = ((mv0, dx0, sx0, ss0), (mv1, dx1, sx1, ss1))
        ebase0 = wid * EPW

        def issue_g(s, t):
            g_, sg_ = gl[t % 3]
            tsl = pl.ds(t * KCH, KCH)
            pltpu.async_copy(gsrc.at[sisc.at[tsl]], g_, sg_)

        def issue_dc(s, t):
            d_, c_, sd_, sc_, se_ = dc[t % 2]
            base = ebase0 + (s * SCH + t) * KCH
            tsl = pl.ds(t * KCH, KCH)
            if write_e:
                def _drain():
                    pltpu.make_async_copy(c_, e_hbm.at[pl.ds(0, KCH)],
                                          se_).wait()
                if t >= 2:
                    _drain()
                else:
                    @pl.when(s > 0)
                    def _():
                        _drain()
            pltpu.async_copy(gdst.at[disc.at[tsl]], d_, sd_)
            pltpu.async_copy(cterm.at[pl.ds(base, KCH)], c_, sc_)

        def process(s, t):
            g_, sg_ = gl[t % 3]
            d_, c_, sd_, sc_, se_ = dc[t % 2]
            mv_, dx_, sx_, ss_ = st[t % 2]
            base = ebase0 + (s * SCH + t) * KCH
            tsl = pl.ds(t * KCH, KCH)

            def _drain_sc():
                pltpu.make_async_copy(mv_, aggsh.at[dx_], ss_).wait()
            if t >= 2:
                _drain_sc()
            else:
                @pl.when(s > 0)
                def _():
                    _drain_sc()
            pltpu.async_copy(dsti.at[pl.ds(base, KCH)], dx_, sx_)

            pltpu.make_async_copy(gsrc.at[sisc.at[tsl]], g_, sg_).wait()
            pltpu.make_async_copy(gdst.at[disc.at[tsl]], d_, sd_).wait()
            pltpu.make_async_copy(cterm.at[pl.ds(base, KCH)], c_, sc_).wait()

            @plsc.parallel_loop(0, KCH, 1, unroll=4)
            def row(j):
                for qq in range(4):
                    slw = pl.ds(16 * qq, 16)
                    a0, a1 = _unpack_words(g_[j, slw])
                    b0 = d_[j, pl.ds(16 * qq, 16)]
                    b1 = d_[j, pl.ds(HW + 16 * qq, 16)]
                    k0, k1 = _unpack_words(c_[j, slw])
                    e0 = jnp.maximum(a0 + b0 + k0, 0.0)
                    e1 = jnp.maximum(a1 + b1 + k1, 0.0)
                    x0, x1 = _unpack_words(g_[j, pl.ds(HW + 16 * qq, 16)])
                    m0 = jnp.maximum(x0 + e0, 0.0)
                    m1 = jnp.maximum(x1 + e1, 0.0)
                    if write_e:
                        c_[j, slw] = _pack_words_fast(e0, e1)
                    mv_[j, pl.ds(32 * qq, 16)] = m0
                    mv_[j, pl.ds(32 * qq + 16, 16)] = m1

            if write_e:
                pltpu.async_copy(c_, e_hbm.at[pl.ds(base, KCH)], se_)
            pltpu.make_async_copy(dsti.at[pl.ds(base, KCH)], dx_, sx_).wait()
            pltpu.async_copy(mv_, aggsh.at[dx_], ss_, add=True)

        def super_chunk(s, carry):
            sbase = ebase0 + s * (SCH * KCH)
            pltpu.sync_copy(srci.at[pl.ds(sbase, SCH * KCH)], sisc)
            pltpu.sync_copy(dsti.at[pl.ds(sbase, SCH * KCH)], disc)
            issue_g(s, 0)
            issue_g(s, 1)
            issue_dc(s, 0)
            for t in range(SCH):
                if t + 2 < SCH:
                    issue_g(s, t + 2)
                if t + 1 < SCH:
                    issue_dc(s, t + 1)
                process(s, t)
            return carry

        lax.fori_loop(0, NCHUNK // SCH, super_chunk, 0)
        if write_e:
            pltpu.make_async_copy(c0, e_hbm.at[pl.ds(0, KCH)], se0).wait()
            pltpu.make_async_copy(c1, e_hbm.at[pl.ds(0, KCH)], se1).wait()
        pltpu.make_async_copy(mv0, aggsh.at[dx0], ss0).wait()
        pltpu.make_async_copy(mv1, aggsh.at[dx1], ss1).wait()
        plsc.subcore_barrier()

        o0 = cid * N
        pltpu.sync_copy(aggsh.at[pl.ds(r0, RPT)], agg_hbm.at[pl.ds(o0 + r0, RPT)])

        @pl.when(sid == NSUB - 1)
        def _():
            pltpu.sync_copy(aggsh.at[pl.ds(TAIL0, TAILN)],
                            agg_hbm.at[pl.ds(o0 + TAIL0, TAILN)])

    return pl.kernel(body, out_type=out_type, mesh=mesh, scratch_types=scratch)


_sc_edge_we = _make_sc_edge(True)
_sc_edge_noe = _make_sc_edge(False)


def kernel(x, edge_index, edge_attr, batch, We0, be0, Wx0, W20, b20,
           We1, be1, Wx1, W21, b21, We2, be2, Wx2, W22, b22, lin_W, lin_b):
    src = edge_index[0]
    dst = edge_index[1]
    zeros = jnp.zeros((N, H), jnp.float32)
    params = [(We0, be0, Wx0, W20, b20), (We1, be1, Wx1, W21, b21),
              (We2, be2, Wx2, W22, b22)]

    tau = jnp.array([16 * (p // 32) + (p % 32) if p % 32 < 16
                     else HW + 16 * (p // 32) + (p % 32 - 16)
                     for p in range(H)], dtype=jnp.int32)

    wsplit = []
    for (We, be, Wx, W2, b2) in params:
        Wa = We[:H]
        Wb = We[H:2 * H]
        Wc = We[2 * H:]
        wsrc = jnp.concatenate([Wa, Wx], axis=1)
        wsplit.append((wsrc, Wb, Wc, be.reshape(1, H),
                       W2[:HW], W2[HW:], W2[tau, :], b2.reshape(1, H)))

    ea = edge_attr
    means = []
    gsrc, gdst = _proj(x, wsplit[0][0], wsplit[0][1])
    for b in range(2):
        wsrc, Wb, Wc, be1h, w2lo, w2hi, w2p, b21h = wsplit[b]
        if b == 0:
            C = _edgemm0(ea, Wc, be1h)
        else:
            C = _edgemmw(ea, Wc[:HW], Wc[HW:], be1h)
        e_new, agg = _sc_edge_we(gsrc, gdst, C, src, dst, zeros)
        gsrc, gdst, mean = _postproj(gsrc, agg, w2lo, w2hi, w2p, b21h,
                                     wsplit[b + 1][0], wsplit[b + 1][1])
        ea = e_new
        means.append(mean)

    wsrc, Wb, Wc, be1h, w2lo, w2hi, w2p, b21h = wsplit[2]
    C = _edgemmw(ea, Wc[:HW], Wc[HW:], be1h)
    (agg,) = _sc_edge_noe(gsrc, gdst, C, src, dst, zeros)
    lw = lin_W.reshape(3, H)
    return _final(gsrc, agg, w2lo, w2hi, w2p, b21h, means[0], means[1], lw,
                  lin_b.reshape(1, 1))
